# baseline (speedup 1.0000x reference)
import jax
import jax.numpy as jnp
from jax import lax
from jax.experimental import pallas as pl
from jax.experimental.pallas import tpu as pltpu
from jax.experimental.pallas import tpu_sc as plsc

BATCH = 16384
HIST = 200
NROW = 100000
DIM = 3
NC, NS = 2, 16
NW = NC * NS
NIDX = BATCH * HIST
BLK = 4096
ROWSTRIDE = BATCH * 8
NIBLK = HIST // 8
NCHK = 4
CHK = NROW // NCHK


def _lookup_body(idx_hbm, ex_hbm, ey_hbm, ez_hbm, out_hbm,
                 px_sh, py_sh, pz_sh, bounce_v,
                 idx_v0, idx_v1, ov00, ov01, ov02, ov10, ov11, ov12,
                 isem0, isem1, gsem0, gsem1, osem0, osem1):
    cid = lax.axis_index("c")
    sid = lax.axis_index("s")
    wid = sid * NC + cid
    base = pl.multiple_of(wid * BLK, BLK)
    isems = (isem0, isem1)
    gsems = (gsem0, gsem1)
    osems = (osem0, osem1)
    planes = (px_sh, py_sh, pz_sh)
    idx_bufs = (idx_v0, idx_v1)
    out_bufs = ((ov00, ov01, ov02), (ov10, ov11, ov12))

    for d, src in enumerate((ex_hbm, ey_hbm, ez_hbm)):
        for c in range(NCHK):
            @pl.when(sid == d * NCHK + c)
            def _():
                o = pl.multiple_of(c * CHK, CHK)
                pltpu.sync_copy(src.at[pl.ds(o, CHK)], bounce_v)
                pltpu.sync_copy(bounce_v, planes[d].at[pl.ds(o, CHK)])
    plsc.subcore_barrier()

    def idx_src(i):
        off = pl.multiple_of(i * ROWSTRIDE + base, BLK)
        return idx_hbm.at[pl.ds(off, BLK)]

    def out_dst(d, i):
        off = pl.multiple_of(d * NIDX + i * ROWSTRIDE + base, BLK)
        return out_hbm.at[pl.ds(off, BLK)]

    def step(i, b, prefetch_i, wait_store):
        if prefetch_i is not None:
            pltpu.async_copy(idx_src(prefetch_i), idx_bufs[1 - b],
                             isems[1 - b])
        pltpu.make_async_copy(idx_src(i), idx_bufs[b], isems[b]).wait()
        for d in range(DIM):
            pltpu.async_copy(planes[d].at[idx_bufs[b]], out_bufs[b][d],
                             gsems[b])
        for d in range(DIM):
            pltpu.make_async_copy(planes[d].at[idx_bufs[b]], out_bufs[b][d],
                                  gsems[b]).wait()
        if wait_store:
            for d in range(DIM):
                pltpu.make_async_copy(out_bufs[b][d], out_dst(d, i),
                                      osems[b]).wait()
        for d in range(DIM):
            pltpu.async_copy(out_bufs[b][d], out_dst(d, i), osems[b])

    pltpu.async_copy(idx_src(0), idx_bufs[0], isems[0])
    step(0, 0, 1, False)
    step(1, 1, 2, False)

    def pair(t, c):
        i0 = pl.multiple_of(2 + 2 * t, 2)
        step(i0, 0, i0 + 1, True)
        step(i0 + 1, 1, i0 + 2, True)
        return c

    lax.fori_loop(0, (NIBLK - 3) // 2, pair, 0, unroll=False)
    step(NIBLK - 1, 0, None, True)

    for d in range(DIM):
        pltpu.make_async_copy(out_bufs[1][d], out_dst(d, NIBLK - 2),
                              osems[1]).wait()
    for d in range(DIM):
        pltpu.make_async_copy(out_bufs[0][d], out_dst(d, NIBLK - 1),
                              osems[0]).wait()


def kernel(indices, embedding):
    idx_flat = (indices.T.reshape(HIST // 8, 8, BATCH // 128, 128)
                .transpose(0, 2, 1, 3).reshape(NIDX))
    ex = embedding[:, 0]
    ey = embedding[:, 1]
    ez = embedding[:, 2]
    mesh = plsc.VectorSubcoreMesh(core_axis_name="c", subcore_axis_name="s")
    out_flat = pl.kernel(
        _lookup_body,
        out_type=jax.ShapeDtypeStruct((DIM * NIDX,), jnp.float32),
        mesh=mesh,
        compiler_params=pltpu.CompilerParams(needs_layout_passes=False),
        scratch_types=[
            pltpu.VMEM_SHARED((NROW,), jnp.float32),
            pltpu.VMEM_SHARED((NROW,), jnp.float32),
            pltpu.VMEM_SHARED((NROW,), jnp.float32),
            pltpu.VMEM((CHK,), jnp.float32),
            pltpu.VMEM((BLK,), jnp.int32),
            pltpu.VMEM((BLK,), jnp.int32),
            pltpu.VMEM((BLK,), jnp.float32),
            pltpu.VMEM((BLK,), jnp.float32),
            pltpu.VMEM((BLK,), jnp.float32),
            pltpu.VMEM((BLK,), jnp.float32),
            pltpu.VMEM((BLK,), jnp.float32),
            pltpu.VMEM((BLK,), jnp.float32),
            pltpu.SemaphoreType.DMA,
            pltpu.SemaphoreType.DMA,
            pltpu.SemaphoreType.DMA,
            pltpu.SemaphoreType.DMA,
            pltpu.SemaphoreType.DMA,
            pltpu.SemaphoreType.DMA,
        ],
    )(idx_flat, ex, ey, ez)
    out_t = (out_flat.reshape(DIM, HIST // 8, BATCH // 128, 8, 128)
             .transpose(0, 1, 3, 2, 4).reshape(DIM, HIST, BATCH))
    return out_t.transpose(2, 1, 0)


# speedup vs baseline: 1.1605x; 1.1605x over previous
import jax
import jax.numpy as jnp
from jax import lax
from jax.experimental import pallas as pl
from jax.experimental.pallas import tpu as pltpu
from jax.experimental.pallas import tpu_sc as plsc

BATCH = 16384
HIST = 200
NROW = 100000
DIM = 3
NC, NS = 2, 16
NW = NC * NS
BCOL = BATCH // NW
HBLK = 8
NIBLK = HIST // HBLK
VEC = 16
KUNROLL = 2
NCHK = 4
CHK = NROW // NCHK


def _unit_compute(idx_b, out_b, plane_v):
    @plsc.parallel_loop(0, BCOL, step=VEC, unroll=KUNROLL)
    def _(col):
        col = pl.multiple_of(col, VEC)
        for s in range(HBLK):
            vidx = idx_b[s, pl.ds(col, VEC)]
            out_b[s, pl.ds(col, VEC)] = plsc.load_gather(plane_v, [vidx])


def _lookup_body(idx_hbm, ex_hbm, ey_hbm, ez_hbm, out_hbm,
                 planes_sh, plane_v, idx_v, out_v,
                 isem0, isem1, osem0, osem1):
    cid = lax.axis_index("c")
    sid = lax.axis_index("s")
    wid = sid * NC + cid
    bcol = pl.multiple_of(wid * BCOL, BCOL)
    isems = (isem0, isem1)
    osems = (osem0, osem1)

    for d, src in enumerate((ey_hbm, ez_hbm)):
        for c in range(NCHK):
            @pl.when(sid == d * NCHK + c)
            def _():
                o = pl.multiple_of(c * CHK, CHK)
                pltpu.sync_copy(src.at[pl.ds(o, CHK)],
                                plane_v.at[pl.ds(0, CHK)])
                pltpu.sync_copy(plane_v.at[pl.ds(0, CHK)],
                                planes_sh.at[pl.ds(d * NROW + o, CHK)])
    plsc.subcore_barrier()

    def idx_src(i):
        r0 = pl.multiple_of(i * HBLK, HBLK)
        return idx_hbm.at[pl.ds(r0, HBLK), pl.ds(bcol, BCOL)]

    def out_dst(d, i):
        r0 = pl.multiple_of(i * HBLK, HBLK)
        return out_hbm.at[d, pl.ds(r0, HBLK), pl.ds(bcol, BCOL)]

    def step(d, i, b, prefetch_i, wait_store, tee_wait):
        if prefetch_i is not None:
            pltpu.async_copy(idx_src(prefetch_i), idx_v.at[1 - b],
                             isems[1 - b])
        pltpu.make_async_copy(idx_src(i), idx_v.at[b], isems[b]).wait()
        if wait_store:
            pltpu.make_async_copy(out_v.at[b], out_dst(d, i), osems[b]).wait()
        _unit_compute(idx_v.at[b], out_v.at[b], plane_v)
        pltpu.async_copy(out_v.at[b], out_dst(d, i), osems[b])

    for d in range(DIM):
        if d == 0:
            pltpu.sync_copy(ex_hbm, plane_v)
        else:
            pltpu.sync_copy(planes_sh.at[pl.ds((d - 1) * NROW, NROW)],
                            plane_v)
        pltpu.async_copy(idx_src(0), idx_v.at[0], isems[0])
        step(d, 0, 0, 1, d > 0, False)
        step(d, 1, 1, 2, d > 0, True)

        def pair(t, c):
            i0 = pl.multiple_of(2 + 2 * t, 2)
            step(d, i0, 0, i0 + 1, True, True)
            step(d, i0 + 1, 1, i0 + 2, True, True)
            return c

        lax.fori_loop(0, (NIBLK - 3) // 2, pair, 0, unroll=False)
        step(d, NIBLK - 1, 0, None, True, False)

    pltpu.make_async_copy(out_v.at[1], out_dst(DIM - 1, NIBLK - 2),
                          osems[1]).wait()
    pltpu.make_async_copy(out_v.at[0], out_dst(DIM - 1, NIBLK - 1),
                          osems[0]).wait()


def kernel(indices, embedding):
    idx_t = indices.T
    ex = embedding[:, 0]
    ey = embedding[:, 1]
    ez = embedding[:, 2]
    mesh = plsc.VectorSubcoreMesh(core_axis_name="c", subcore_axis_name="s")
    out_t = pl.kernel(
        _lookup_body,
        out_type=jax.ShapeDtypeStruct((DIM, HIST, BATCH), jnp.float32),
        mesh=mesh,
        compiler_params=pltpu.CompilerParams(needs_layout_passes=False),
        scratch_types=[
            pltpu.VMEM_SHARED(((DIM - 1) * NROW,), jnp.float32),
            pltpu.VMEM((NROW,), jnp.float32),
            pltpu.VMEM((2, HBLK, BCOL), jnp.int32),
            pltpu.VMEM((2, HBLK, BCOL), jnp.float32),
            pltpu.SemaphoreType.DMA,
            pltpu.SemaphoreType.DMA,
            pltpu.SemaphoreType.DMA,
            pltpu.SemaphoreType.DMA,
        ],
    )(idx_t, ex, ey, ez)
    return out_t.transpose(2, 1, 0)
